# baseline (speedup 1.0000x reference)
import functools

import jax
import jax.numpy as jnp
from jax import lax
from jax.experimental import pallas as pl
from jax.experimental.pallas import tpu as pltpu
from jax.experimental.pallas import tpu_sc as plsc

@functools.lru_cache(maxsize=None)
def _sc_gather(B: int, num_steps: int):
    info = plsc.get_sparse_core_info()
    nc, ns, L = info.num_cores, info.num_subcores, info.num_lanes
    nw = nc * ns
    b_per_w = B // nw
    mesh = plsc.VectorSubcoreMesh(core_axis_name="c", subcore_axis_name="s")

    @functools.partial(
        pl.kernel,
        mesh=mesh,
        out_type=jax.ShapeDtypeStruct((2, B), jnp.float32),
        scratch_types=[
            pltpu.VMEM((2 * num_steps,), jnp.float32),
            pltpu.VMEM((b_per_w,), jnp.int32),
            pltpu.VMEM((b_per_w,), jnp.float32),
            pltpu.VMEM((b_per_w,), jnp.float32),
            pltpu.SemaphoreType.DMA,
            pltpu.SemaphoreType.DMA,
        ],
        compiler_params=pltpu.CompilerParams(needs_layout_passes=False),
    )
    def gather(t_hbm, sab_hbm, somab_hbm, out_hbm, tab_v, idx_v, sa_v, sb_v, s0, s1):
        wid = lax.axis_index("s") * nc + lax.axis_index("c")
        base = wid * b_per_w
        ca = pltpu.make_async_copy(sab_hbm, tab_v.at[pl.ds(0, num_steps)], s0)
        cb = pltpu.make_async_copy(somab_hbm, tab_v.at[pl.ds(num_steps, num_steps)], s0)
        cidx = pltpu.make_async_copy(t_hbm.at[pl.ds(base, b_per_w)], idx_v, s1)
        ca.start()
        cb.start()
        cidx.start()
        ca.wait()
        cb.wait()
        cidx.wait()
        for j in range(b_per_w // L):
            idx = idx_v[pl.ds(j * L, L)]
            idx = jnp.minimum(jnp.maximum(idx, 0), num_steps - 1)
            sa_v[pl.ds(j * L, L)] = plsc.load_gather(tab_v, [idx])
            sb_v[pl.ds(j * L, L)] = plsc.load_gather(tab_v, [idx + num_steps])
        ca = pltpu.make_async_copy(sa_v, out_hbm.at[0, pl.ds(base, b_per_w)], s0)
        cb = pltpu.make_async_copy(sb_v, out_hbm.at[1, pl.ds(base, b_per_w)], s1)
        ca.start()
        cb.start()
        ca.wait()
        cb.wait()

    return gather


def _tc_combine_body(sc_ref, x_ref, n_ref, o_ref):
    o_ref[...] = (sc_ref[0:1, :] * x_ref[...]
                  + sc_ref[1:2, :] * n_ref[...])


@functools.lru_cache(maxsize=None)
def _tc_combine(F: int, B: int, R: int):
    data = pl.BlockSpec((R, B), lambda i: (i, 0))
    scale = pl.BlockSpec((2, B), lambda i: (0, 0))
    shape = jax.ShapeDtypeStruct((F, B), jnp.float32)
    return pl.pallas_call(
        _tc_combine_body,
        grid=(F // R,),
        in_specs=[scale, data, data],
        out_specs=data,
        out_shape=shape,
    )


def kernel(x_start, noise, t, sqrt_alpha_bars, sqrt_one_minus_alpha_bars):
    B, C, H, W = x_start.shape
    F = C * H * W
    num_steps = sqrt_alpha_bars.shape[0]
    scales = _sc_gather(B, num_steps)(
        t, sqrt_alpha_bars, sqrt_one_minus_alpha_bars)
    xT = jnp.transpose(x_start, (1, 2, 3, 0)).reshape(F, B)
    nT = jnp.transpose(noise, (1, 2, 3, 0)).reshape(F, B)
    oT = _tc_combine(F, B, 2048)(scales, xT, nT)
    x_t = jnp.transpose(oT.reshape(C, H, W, B), (3, 0, 1, 2))
    return (x_t, noise)


# speedup vs baseline: 1.0108x; 1.0108x over previous
import functools

import jax
import jax.numpy as jnp
from jax import lax
from jax.experimental import pallas as pl
from jax.experimental.pallas import tpu as pltpu
from jax.experimental.pallas import tpu_sc as plsc

@functools.lru_cache(maxsize=None)
def _sc_gather(B: int, num_steps: int):
    info = plsc.get_sparse_core_info()
    nc, ns, L = info.num_cores, info.num_subcores, info.num_lanes
    nw = nc * ns
    b_per_w = B // nw
    mesh = plsc.VectorSubcoreMesh(core_axis_name="c", subcore_axis_name="s")

    @functools.partial(
        pl.kernel,
        mesh=mesh,
        out_type=jax.ShapeDtypeStruct((2, B), jnp.float32),
        scratch_types=[
            pltpu.VMEM((2 * num_steps,), jnp.float32),
            pltpu.VMEM((b_per_w,), jnp.int32),
            pltpu.VMEM((b_per_w,), jnp.float32),
            pltpu.VMEM((b_per_w,), jnp.float32),
            pltpu.SemaphoreType.DMA,
            pltpu.SemaphoreType.DMA,
        ],
        compiler_params=pltpu.CompilerParams(needs_layout_passes=False),
    )
    def gather(t_hbm, sab_hbm, somab_hbm, out_hbm, tab_v, idx_v, sa_v, sb_v, s0, s1):
        wid = lax.axis_index("s") * nc + lax.axis_index("c")
        base = wid * b_per_w
        ca = pltpu.make_async_copy(sab_hbm, tab_v.at[pl.ds(0, num_steps)], s0)
        cb = pltpu.make_async_copy(somab_hbm, tab_v.at[pl.ds(num_steps, num_steps)], s0)
        cidx = pltpu.make_async_copy(t_hbm.at[pl.ds(base, b_per_w)], idx_v, s1)
        ca.start()
        cb.start()
        cidx.start()
        ca.wait()
        cb.wait()
        cidx.wait()
        for j in range(b_per_w // L):
            idx = idx_v[pl.ds(j * L, L)]
            idx = jnp.minimum(jnp.maximum(idx, 0), num_steps - 1)
            sa_v[pl.ds(j * L, L)] = plsc.load_gather(tab_v, [idx])
            sb_v[pl.ds(j * L, L)] = plsc.load_gather(tab_v, [idx + num_steps])
        ca = pltpu.make_async_copy(sa_v, out_hbm.at[0, pl.ds(base, b_per_w)], s0)
        cb = pltpu.make_async_copy(sb_v, out_hbm.at[1, pl.ds(base, b_per_w)], s1)
        ca.start()
        cb.start()
        ca.wait()
        cb.wait()

    return gather


def _tc_combine_body(sc_ref, x_ref, n_ref, o_ref):
    o_ref[...] = (sc_ref[0:1, :] * x_ref[...]
                  + sc_ref[1:2, :] * n_ref[...])


@functools.lru_cache(maxsize=None)
def _tc_combine(F: int, B: int, R: int):
    data = pl.BlockSpec((R, B), lambda i: (i, 0))
    scale = pl.BlockSpec((2, B), lambda i: (0, 0))
    shape = jax.ShapeDtypeStruct((F, B), jnp.float32)
    return pl.pallas_call(
        _tc_combine_body,
        grid=(F // R,),
        in_specs=[scale, data, data],
        out_specs=data,
        out_shape=shape,
    )


def kernel(x_start, noise, t, sqrt_alpha_bars, sqrt_one_minus_alpha_bars):
    B, C, H, W = x_start.shape
    F = C * H * W
    num_steps = sqrt_alpha_bars.shape[0]
    scales = _sc_gather(B, num_steps)(
        t, sqrt_alpha_bars, sqrt_one_minus_alpha_bars)
    xT = jnp.transpose(x_start, (1, 2, 3, 0)).reshape(F, B)
    nT = jnp.transpose(noise, (1, 2, 3, 0)).reshape(F, B)
    oT = _tc_combine(F, B, 1024)(scales, xT, nT)
    x_t = jnp.transpose(oT.reshape(C, H, W, B), (3, 0, 1, 2))
    return (x_t, noise)
